# baseline (device time: 41817 ns/iter reference)
import jax
import jax.numpy as jnp
from jax import lax
from jax.experimental import pallas as pl
from jax.experimental.pallas import tpu as pltpu


def kernel(x, assign, W1, W2):
    t, d = x.shape
    e, _, f = W1.shape
    assign2d = assign.reshape(t, 1)

    def body(x_ref, a_ref, w1_ref, w2_ref, out_ref,
             x_recv, a_recv, part_theirs, part_recv, sems):
        my_x = lax.axis_index("x")
        my_y = lax.axis_index("y")
        my_z = lax.axis_index("z")
        peer = (my_x, 1 - my_y, my_z)

        barrier_sem = pltpu.get_barrier_semaphore()
        pl.semaphore_signal(barrier_sem, inc=1, device_id=peer,
                            device_id_type=pl.DeviceIdType.MESH)
        pl.semaphore_wait(barrier_sem, 1)

        rdma_x = pltpu.make_async_remote_copy(
            src_ref=x_ref, dst_ref=x_recv,
            send_sem=sems.at[0], recv_sem=sems.at[1],
            device_id=peer, device_id_type=pl.DeviceIdType.MESH)
        rdma_a = pltpu.make_async_remote_copy(
            src_ref=a_ref, dst_ref=a_recv,
            send_sem=sems.at[2], recv_sem=sems.at[3],
            device_id=peer, device_id_type=pl.DeviceIdType.MESH)
        rdma_x.start()
        rdma_a.start()

        def moe_local_experts(xv, av):
            acc = jnp.zeros((t, d), jnp.float32)
            for l in range(e):
                gid = e * my_y + l
                xm = jnp.where(av == gid, xv, 0.0)
                h = jnp.maximum(
                    jnp.dot(xm, w1_ref[l], preferred_element_type=jnp.float32),
                    0.0)
                acc = acc + jnp.dot(
                    h, w2_ref[l], preferred_element_type=jnp.float32)
            return acc

        out_mine = moe_local_experts(x_ref[...], a_ref[...])

        rdma_x.wait()
        rdma_a.wait()

        part_theirs[...] = moe_local_experts(x_recv[...], a_recv[...])

        rdma_p = pltpu.make_async_remote_copy(
            src_ref=part_theirs, dst_ref=part_recv,
            send_sem=sems.at[4], recv_sem=sems.at[5],
            device_id=peer, device_id_type=pl.DeviceIdType.MESH)
        rdma_p.start()
        rdma_p.wait()

        out_ref[...] = out_mine + part_recv[...]

    return pl.pallas_call(
        body,
        out_shape=jax.ShapeDtypeStruct((t, d), jnp.float32),
        in_specs=[pl.BlockSpec(memory_space=pltpu.VMEM)] * 4,
        out_specs=pl.BlockSpec(memory_space=pltpu.VMEM),
        scratch_shapes=[
            pltpu.VMEM((t, d), jnp.float32),
            pltpu.VMEM((t, 1), jnp.int32),
            pltpu.VMEM((t, d), jnp.float32),
            pltpu.VMEM((t, d), jnp.float32),
            pltpu.SemaphoreType.DMA((6,)),
        ],
        compiler_params=pltpu.CompilerParams(collective_id=0),
    )(x, assign2d, W1, W2)
